# Initial kernel scaffold; baseline (speedup 1.0000x reference)
import jax
import jax.numpy as jnp
from jax.experimental import pallas as pl


def kernel(pos, remap_node_type, node_emb_W, node_emb_b, edge_emb, l0_We1, l0_be1, l0_We2, l0_be2, l0_Wu1, l0_bu1, l0_Wu2, l0_bu2, l1_We1, l1_be1, l1_We2, l1_be2, l1_Wu1, l1_bu1, l1_Wu2, l1_bu2, Wo1, bo1, Wo2, bo2, node_type, edge_index, edge_type, batch):
    raise NotImplementedError("write your pallas kernel here")


# speedup vs baseline: 3.5278x; 3.5278x over previous
import functools

import jax
import jax.numpy as jnp
from jax import lax
from jax.experimental import pallas as pl
from jax.experimental.pallas import tpu as pltpu
from jax.experimental.pallas import tpu_sc as plsc

_N = 10000
_E = 320000
_H = 128
_NG = 50
_B = 64
_R_MAX = 10.0
_MAX_CHARGE = 9.0

_NC, _NS = 2, 16
_NW = _NC * _NS
_CH = 128
_NCHUNK = _E // _CH
_CPW = _NCHUNK // _NW
_REM = _NCHUNK % _NW

_PREC = lax.Precision.HIGHEST


def _vsmesh():
    return plsc.VectorSubcoreMesh(core_axis_name="c", subcore_axis_name="s",
                                  num_cores=_NC, num_subcores=_NS)


def _sc_prep(pxa, pya, pza, src, dst, zeros_n8, ones_c8):
    @functools.partial(
        pl.kernel,
        out_type=[
            jax.ShapeDtypeStruct((_E,), jnp.float32),
            jax.ShapeDtypeStruct((_NC, _N, 8), jnp.float32),
        ],
        mesh=_vsmesh(),
        scratch_types=[
            pltpu.VMEM((_N,), jnp.float32),
            pltpu.VMEM((_N,), jnp.float32),
            pltpu.VMEM((_N,), jnp.float32),
            pltpu.VMEM((_CH,), jnp.int32),
            pltpu.VMEM((_CH,), jnp.int32),
            pltpu.VMEM((_CH,), jnp.float32),
            pltpu.VMEM((_CH, 8), jnp.float32),
            pltpu.VMEM_SHARED((_N, 8), jnp.float32),
        ],
        compiler_params=pltpu.CompilerParams(needs_layout_passes=False),
    )
    def body(px_ref, py_ref, pz_ref, src_ref, dst_ref, z8_ref, ones_ref,
             d2_ref, deg_ref, px, py, pz, srcv, dstv, d2v, onesv, dacc):
        cid = lax.axis_index("c")
        sid = lax.axis_index("s")
        wid = sid * _NC + cid
        pltpu.sync_copy(px_ref, px)
        pltpu.sync_copy(py_ref, py)
        pltpu.sync_copy(pz_ref, pz)
        pltpu.sync_copy(ones_ref, onesv)

        @pl.when(sid == 0)
        def _():
            pltpu.sync_copy(z8_ref, dacc)

        plsc.subcore_barrier()

        def chunk(g, carry):
            c = wid + g * _NW
            base = c * _CH
            pltpu.sync_copy(src_ref.at[pl.ds(base, _CH)], srcv)
            pltpu.sync_copy(dst_ref.at[pl.ds(base, _CH)], dstv)

            def grp(k, carry2):
                sl = pl.ds(k * 16, 16)
                si = srcv[sl]
                di = dstv[sl]
                dx = plsc.load_gather(px, [di]) - plsc.load_gather(px, [si])
                dy = plsc.load_gather(py, [di]) - plsc.load_gather(py, [si])
                dz = plsc.load_gather(pz, [di]) - plsc.load_gather(pz, [si])
                d2v[sl] = dx * dx + dy * dy + dz * dz + 1e-8
                return carry2

            lax.fori_loop(0, _CH // 16, grp, 0, unroll=4)
            pltpu.sync_copy(d2v, d2_ref.at[pl.ds(base, _CH)])
            pltpu.sync_copy(onesv, dacc.at[dstv], add=True)
            return carry

        nch = _CPW + jnp.where(wid < _REM, 1, 0)
        lax.fori_loop(0, nch, chunk, 0)
        plsc.subcore_barrier()

        @pl.when(sid == 0)
        def _():
            pltpu.sync_copy(dacc, deg_ref.at[cid])

    return body(pxa, pya, pza, src, dst, zeros_n8, ones_c8)


def _sc_layer(hs, hd, rwt, src, dst, zeros_nh):
    @functools.partial(
        pl.kernel,
        out_type=jax.ShapeDtypeStruct((_NC, _N, _H), jnp.float32),
        mesh=_vsmesh(),
        scratch_types=[
            pltpu.VMEM((_CH,), jnp.int32),
            pltpu.VMEM((_CH,), jnp.int32),
            pltpu.VMEM((_CH, _H), jnp.float32),
            pltpu.VMEM((_CH, _H), jnp.float32),
            pltpu.VMEM((_CH, _H), jnp.float32),
            pltpu.VMEM_SHARED((_N, _H), jnp.float32),
            pltpu.SemaphoreType.DMA,
            pltpu.SemaphoreType.DMA,
            pltpu.SemaphoreType.DMA,
        ],
    )
    def body(hs_ref, hd_ref, rwt_ref, src_ref, dst_ref, z_ref, out_ref,
             srcv, dstv, bufa, bufb, bufc, acc, sema, semb, semc):
        cid = lax.axis_index("c")
        sid = lax.axis_index("s")
        wid = sid * _NC + cid

        @pl.when(sid == 0)
        def _():
            pltpu.sync_copy(z_ref, acc)

        plsc.subcore_barrier()

        def chunk(g, carry):
            c = wid + g * _NW
            base = c * _CH
            pltpu.sync_copy(src_ref.at[pl.ds(base, _CH)], srcv)
            pltpu.sync_copy(dst_ref.at[pl.ds(base, _CH)], dstv)
            cpa = pltpu.async_copy(hs_ref.at[srcv], bufa, sema)
            cpb = pltpu.async_copy(hd_ref.at[dstv], bufb, semb)
            cpc = pltpu.async_copy(rwt_ref.at[pl.ds(base, _CH), :], bufc, semc)
            cpa.wait()
            cpb.wait()
            cpc.wait()

            def row(r, carry2):
                for gg in range(_H // 16):
                    sl = pl.ds(gg * 16, 16)
                    z = bufa[r, sl] + bufb[r, sl] + bufc[r, sl]
                    bufa[r, sl] = jnp.maximum(z, 0.0)
                return carry2

            lax.fori_loop(0, _CH, row, 0)
            pltpu.sync_copy(bufa, acc.at[dstv], add=True)
            return carry

        nch = _CPW + jnp.where(wid < _REM, 1, 0)
        lax.fori_loop(0, nch, chunk, 0)
        plsc.subcore_barrier()

        @pl.when(sid == 0)
        def _():
            pltpu.sync_copy(acc, out_ref.at[cid])

    return body(hs, hd, rwt, src, dst, zeros_nh)


_EBLK = 3200


def _tc_edge(d2, etype_f, wr_cat, t_cat):
    def body(d2_ref, et_ref, wr_ref, t_ref, o0_ref, o1_ref):
        d = jnp.sqrt(d2_ref[...])
        width = _R_MAX / (_NG - 1)
        mu = lax.broadcasted_iota(jnp.int32, (_EBLK, _NG), 1).astype(
            jnp.float32) * width
        u = (d - mu) * (1.0 / width)
        rf = jnp.exp(-0.5 * u * u)
        onehot = (et_ref[...] ==
                  lax.broadcasted_iota(jnp.int32, (_EBLK, 4), 1)
                  ).astype(jnp.float32)
        rwt = (jnp.dot(rf, wr_ref[...], precision=_PREC,
                       preferred_element_type=jnp.float32) +
               jnp.dot(onehot, t_ref[...], precision=_PREC,
                       preferred_element_type=jnp.float32))
        o0_ref[...] = rwt[:, :_H]
        o1_ref[...] = rwt[:, _H:]

    return pl.pallas_call(
        body,
        grid=(_E // _EBLK,),
        in_specs=[
            pl.BlockSpec((_EBLK, 1), lambda i: (i, 0)),
            pl.BlockSpec((_EBLK, 1), lambda i: (i, 0)),
            pl.BlockSpec((_NG, 2 * _H), lambda i: (0, 0)),
            pl.BlockSpec((4, 2 * _H), lambda i: (0, 0)),
        ],
        out_specs=[
            pl.BlockSpec((_EBLK, _H), lambda i: (i, 0)),
            pl.BlockSpec((_EBLK, _H), lambda i: (i, 0)),
        ],
        out_shape=[
            jax.ShapeDtypeStruct((_E, _H), jnp.float32),
            jax.ShapeDtypeStruct((_E, _H), jnp.float32),
        ],
    )(d2, etype_f, wr_cat, t_cat)


_NBLK = 2000


def _tc_node_embed(remap, charge1, emb_w, emb_b, wsrc, wdst):
    def body(rm_ref, c1_ref, w_ref, b_ref, ws_ref, wd_ref,
             h_ref, hs_ref, hd_ref):
        c1 = c1_ref[...]
        cps = [jnp.ones_like(c1), c1, c1 * c1]
        h = jnp.zeros((_NBLK, _H), jnp.float32) + b_ref[...]
        for t in range(5):
            rmt = rm_ref[:, t:t + 1]
            for p in range(3):
                h = h + (rmt * cps[p]) * w_ref[3 * t + p:3 * t + p + 1, :]
        h_ref[...] = h
        hs_ref[...] = jnp.dot(h, ws_ref[...], precision=_PREC,
                              preferred_element_type=jnp.float32)
        hd_ref[...] = jnp.dot(h, wd_ref[...], precision=_PREC,
                              preferred_element_type=jnp.float32)

    return pl.pallas_call(
        body,
        grid=(_N // _NBLK,),
        in_specs=[
            pl.BlockSpec((_NBLK, 5), lambda i: (i, 0)),
            pl.BlockSpec((_NBLK, 1), lambda i: (i, 0)),
            pl.BlockSpec((15, _H), lambda i: (0, 0)),
            pl.BlockSpec((1, _H), lambda i: (0, 0)),
            pl.BlockSpec((_H, _H), lambda i: (0, 0)),
            pl.BlockSpec((_H, _H), lambda i: (0, 0)),
        ],
        out_specs=[
            pl.BlockSpec((_NBLK, _H), lambda i: (i, 0)),
            pl.BlockSpec((_NBLK, _H), lambda i: (i, 0)),
            pl.BlockSpec((_NBLK, _H), lambda i: (i, 0)),
        ],
        out_shape=[
            jax.ShapeDtypeStruct((_N, _H), jnp.float32),
            jax.ShapeDtypeStruct((_N, _H), jnp.float32),
            jax.ShapeDtypeStruct((_N, _H), jnp.float32),
        ],
    )(remap, charge1, emb_w, emb_b, wsrc, wdst)


def _node_update(h, agg0, agg1, deg, we2, be2, wu1h, wu1a, bu1, wu2, bu2):
    aggz = agg0 + agg1
    agg = (jnp.dot(aggz, we2, precision=_PREC,
                   preferred_element_type=jnp.float32) + deg * be2)
    u = jnp.maximum(
        jnp.dot(h, wu1h, precision=_PREC, preferred_element_type=jnp.float32)
        + jnp.dot(agg, wu1a, precision=_PREC,
                  preferred_element_type=jnp.float32) + bu1, 0.0)
    upd = jnp.dot(u, wu2, precision=_PREC,
                  preferred_element_type=jnp.float32) + bu2
    return h + upd


def _tc_node_mid(h, agg0, agg1, deg, we2, be2, wu1h, wu1a, bu1, wu2, bu2,
                 wsrc, wdst):
    def body(h_ref, a0_ref, a1_ref, deg_ref, we2_ref, be2_ref, wu1h_ref,
             wu1a_ref, bu1_ref, wu2_ref, bu2_ref, ws_ref, wd_ref,
             hn_ref, hs_ref, hd_ref):
        hn = _node_update(h_ref[...], a0_ref[...], a1_ref[...], deg_ref[...],
                          we2_ref[...], be2_ref[...], wu1h_ref[...],
                          wu1a_ref[...], bu1_ref[...], wu2_ref[...],
                          bu2_ref[...])
        hn_ref[...] = hn
        hs_ref[...] = jnp.dot(hn, ws_ref[...], precision=_PREC,
                              preferred_element_type=jnp.float32)
        hd_ref[...] = jnp.dot(hn, wd_ref[...], precision=_PREC,
                              preferred_element_type=jnp.float32)

    full = lambda *shape: pl.BlockSpec(shape, lambda i: (0,) * len(shape))
    blk = pl.BlockSpec((_NBLK, _H), lambda i: (i, 0))
    return pl.pallas_call(
        body,
        grid=(_N // _NBLK,),
        in_specs=[blk, blk, blk,
                  pl.BlockSpec((_NBLK, 1), lambda i: (i, 0)),
                  full(_H, _H), full(1, _H), full(_H, _H), full(_H, _H),
                  full(1, _H), full(_H, _H), full(1, _H),
                  full(_H, _H), full(_H, _H)],
        out_specs=[blk, blk, blk],
        out_shape=[
            jax.ShapeDtypeStruct((_N, _H), jnp.float32),
            jax.ShapeDtypeStruct((_N, _H), jnp.float32),
            jax.ShapeDtypeStruct((_N, _H), jnp.float32),
        ],
    )(h, agg0, agg1, deg, we2, be2, wu1h, wu1a, bu1, wu2, bu2, wsrc, wdst)


def _tc_node_out(h, agg0, agg1, deg, we2, be2, wu1h, wu1a, bu1, wu2, bu2,
                 wo1, bo1, wo2, bo2, batch_r):
    def body(h_ref, a0_ref, a1_ref, deg_ref, we2_ref, be2_ref, wu1h_ref,
             wu1a_ref, bu1_ref, wu2_ref, bu2_ref, wo1_ref, bo1_ref,
             wo2_ref, bo2_ref, b_ref, out_ref):
        hn = _node_update(h_ref[...], a0_ref[...], a1_ref[...], deg_ref[...],
                          we2_ref[...], be2_ref[...], wu1h_ref[...],
                          wu1a_ref[...], bu1_ref[...], wu2_ref[...],
                          bu2_ref[...])
        ho = jnp.maximum(
            jnp.dot(hn, wo1_ref[...], precision=_PREC,
                    preferred_element_type=jnp.float32) + bo1_ref[...], 0.0)
        hout = jnp.dot(ho, wo2_ref[...], precision=_PREC,
                       preferred_element_type=jnp.float32) + bo2_ref[...]
        brow = b_ref[...].reshape(1, _NBLK)
        onehot_t = (lax.broadcasted_iota(jnp.int32, (_B, _NBLK), 0) ==
                    brow).astype(jnp.float32)
        contrib = jnp.dot(onehot_t, hout, precision=_PREC,
                          preferred_element_type=jnp.float32)

        @pl.when(pl.program_id(0) == 0)
        def _():
            out_ref[...] = jnp.zeros_like(out_ref)

        out_ref[...] += contrib

    full = lambda *shape: pl.BlockSpec(shape, lambda i: (0,) * len(shape))
    blk = pl.BlockSpec((_NBLK, _H), lambda i: (i, 0))
    return pl.pallas_call(
        body,
        grid=(_N // _NBLK,),
        in_specs=[blk, blk, blk,
                  pl.BlockSpec((_NBLK, 1), lambda i: (i, 0)),
                  full(_H, _H), full(1, _H), full(_H, _H), full(_H, _H),
                  full(1, _H), full(_H, _H), full(1, _H),
                  full(_H, _H), full(1, _H), full(_H, 1), full(1, 1),
                  pl.BlockSpec((1, 1, _NBLK), lambda i: (i, 0, 0))],
        out_specs=pl.BlockSpec((_B, 1), lambda i: (0, 0)),
        out_shape=jax.ShapeDtypeStruct((_B, 1), jnp.float32),
    )(h, agg0, agg1, deg, we2, be2, wu1h, wu1a, bu1, wu2, bu2,
      wo1, bo1, wo2, bo2, batch_r)


def kernel(pos, remap_node_type, node_emb_W, node_emb_b, edge_emb,
           l0_We1, l0_be1, l0_We2, l0_be2, l0_Wu1, l0_bu1, l0_Wu2, l0_bu2,
           l1_We1, l1_be1, l1_We2, l1_be2, l1_Wu1, l1_bu1, l1_Wu2, l1_bu2,
           Wo1, bo1, Wo2, bo2, node_type, edge_index, edge_type, batch):
    f32 = jnp.float32
    src = edge_index[0].astype(jnp.int32)
    dst = edge_index[1].astype(jnp.int32)

    def split_we1(we1):
        return (we1[:_H], we1[_H:2 * _H], we1[2 * _H:2 * _H + _NG],
                we1[2 * _H + _NG:])

    ws0, wd0, wr0, wew0 = split_we1(l0_We1)
    ws1, wd1, wr1, wew1 = split_we1(l1_We1)
    wr_cat = jnp.concatenate([wr0, wr1], axis=1)
    t_cat = jnp.concatenate([edge_emb @ wew0 + l0_be1,
                             edge_emb @ wew1 + l1_be1], axis=1)

    zeros_nh = jnp.zeros((_N, _H), f32)
    zeros_n8 = jnp.zeros((_N, 8), f32)
    ones_c8 = jnp.ones((_CH, 8), f32)
    pos_t = pos.T.astype(f32)
    pxa, pya, pza = pos_t[0], pos_t[1], pos_t[2]

    d2, degout = _sc_prep(pxa, pya, pza, src, dst, zeros_n8, ones_c8)
    deg = (degout[0, :, 0] + degout[1, :, 0]).reshape(_N, 1)

    etype_f = edge_type.astype(jnp.int32).reshape(_E, 1)
    rwt0, rwt1 = _tc_edge(d2.reshape(_E, 1), etype_f, wr_cat, t_cat)

    charge1 = (node_type.astype(f32) / _MAX_CHARGE).reshape(_N, 1)
    h0, hs0, hd0 = _tc_node_embed(remap_node_type, charge1, node_emb_W,
                                  node_emb_b.reshape(1, _H), ws0, wd0)

    acc0 = _sc_layer(hs0, hd0, rwt0, src, dst, zeros_nh)
    h1, hs1, hd1 = _tc_node_mid(
        h0, acc0[0], acc0[1], deg, l0_We2, l0_be2.reshape(1, _H),
        l0_Wu1[:_H], l0_Wu1[_H:], l0_bu1.reshape(1, _H), l0_Wu2,
        l0_bu2.reshape(1, _H), ws1, wd1)

    acc1 = _sc_layer(hs1, hd1, rwt1, src, dst, zeros_nh)
    batch_r = batch.astype(jnp.int32).reshape(_N // _NBLK, 1, _NBLK)
    out = _tc_node_out(
        h1, acc1[0], acc1[1], deg, l1_We2, l1_be2.reshape(1, _H),
        l1_Wu1[:_H], l1_Wu1[_H:], l1_bu1.reshape(1, _H), l1_Wu2,
        l1_bu2.reshape(1, _H), Wo1, bo1.reshape(1, _H), Wo2,
        bo2.reshape(1, 1), batch_r)

    return (out, pos)
